# Initial kernel scaffold; baseline (speedup 1.0000x reference)
import jax
import jax.numpy as jnp
from jax.experimental import pallas as pl


def kernel(gesture_labels):
    raise NotImplementedError("write your pallas kernel here")


# speedup vs baseline: 5.2534x; 5.2534x over previous
import functools

import jax
import jax.numpy as jnp
from jax import lax
from jax.experimental import pallas as pl
from jax.experimental.pallas import tpu as pltpu
from jax.experimental.pallas import tpu_sc as plsc

T = 4096
L = 16
NCHUNK = T // L
BIG = T + 10
PAD = 8
BUF = PAD + T + PAD


def _sc_body(in_hbm, out_hbm, press_v, rel_v, e_v, out_v):
    cid = lax.axis_index("c")
    sid = lax.axis_index("s")
    w = sid * 2 + cid

    pltpu.sync_copy(in_hbm.at[2 * w], press_v.at[pl.ds(PAD, T)])
    pltpu.sync_copy(in_hbm.at[2 * w + 1], rel_v.at[pl.ds(PAD, T)])

    lane = lax.iota(jnp.int32, L)

    h = press_v[pl.ds(0, L)]
    press_v[pl.ds(0, L)] = jnp.where(lane < PAD, 0.0, h)
    tl = rel_v[pl.ds(T, L)]
    rel_v[pl.ds(T, L)] = jnp.where(lane >= PAD, 0.0, tl)

    def bwd(i, carry):
        base = (NCHUNK - 1 - i) * L
        cur = rel_v[pl.ds(PAD + base, L)]
        nxt = rel_v[pl.ds(PAD + base + 1, L)]
        on = nxt > cur
        negpos = jnp.where(on, -(base + 1) - lane, -BIG)
        sm = jnp.flip(plsc.cummax(jnp.flip(negpos, 0)), 0)
        comb = jnp.maximum(sm, carry)
        end = jnp.minimum(8 - comb, T)
        pcur = press_v[pl.ds(PAD + base, L)]
        pprev = press_v[pl.ds(PAD + base - 1, L)]
        pon = pcur > pprev
        e_v[pl.ds(base, L)] = jnp.where(pon, end, 0)
        return jnp.maximum(carry, jnp.max(negpos))

    lax.fori_loop(0, NCHUNK, bwd, jnp.int32(-BIG))

    def fwd(i, carry):
        base = i * L
        e = e_v[pl.ds(base, L)]
        comb = jnp.maximum(plsc.cummax(e), carry)
        out_v[pl.ds(base, L)] = jnp.where(comb > base + lane, 1.0, 0.0)
        return jnp.maximum(carry, jnp.max(e))

    lax.fori_loop(0, NCHUNK, fwd, jnp.int32(0))

    pltpu.sync_copy(out_v, out_hbm.at[w])


@jax.jit
def _run(x):
    mesh = plsc.VectorSubcoreMesh(core_axis_name="c", subcore_axis_name="s")
    f = pl.kernel(
        _sc_body,
        out_type=jax.ShapeDtypeStruct((32, T), jnp.float32),
        mesh=mesh,
        scratch_types=[
            pltpu.VMEM((BUF,), jnp.float32),
            pltpu.VMEM((BUF,), jnp.float32),
            pltpu.VMEM((T,), jnp.int32),
            pltpu.VMEM((T,), jnp.float32),
        ],
        compiler_params=pltpu.CompilerParams(
            needs_layout_passes=False, use_tc_tiling_on_sc=False
        ),
    )
    return f(x)


def kernel(gesture_labels):
    x = gesture_labels.reshape(64, T)
    return _run(x).reshape(16, 2, T)


# speedup vs baseline: 6.4847x; 1.2344x over previous
import functools

import jax
import jax.numpy as jnp
from jax import lax
from jax.experimental import pallas as pl
from jax.experimental.pallas import tpu as pltpu
from jax.experimental.pallas import tpu_sc as plsc

T = 4096
L = 16
NCHUNK = T // L
BIG = T + 10
PAD = 8
BUF = PAD + T + PAD


def _sc_body(in_hbm, out_hbm, press_v, rel_v, e_v, out_v):
    cid = lax.axis_index("c")
    sid = lax.axis_index("s")
    w = sid * 2 + cid

    pltpu.sync_copy(in_hbm.at[2 * w], press_v.at[pl.ds(PAD, T)])
    pltpu.sync_copy(in_hbm.at[2 * w + 1], rel_v.at[pl.ds(PAD, T)])

    lane = lax.iota(jnp.int32, L)

    h = press_v[pl.ds(0, L)]
    press_v[pl.ds(0, L)] = jnp.where(lane < PAD, 0.0, h)
    tl = rel_v[pl.ds(T, L)]
    rel_v[pl.ds(T, L)] = jnp.where(lane >= PAD, 0.0, tl)

    @plsc.parallel_loop(0, NCHUNK, unroll=8, carry=jnp.int32(-BIG))
    def _bwd(i, carry):
        base = (NCHUNK - 1 - i) * L
        cur = rel_v[pl.ds(PAD + base, L)]
        nxt = rel_v[pl.ds(PAD + base + 1, L)]
        on = nxt > cur
        negpos = jnp.where(on, -(base + 1) - lane, -BIG)
        sm = jnp.flip(plsc.cummax(jnp.flip(negpos, 0)), 0)
        comb = jnp.maximum(sm, carry)
        end = jnp.minimum(8 - comb, T)
        pcur = press_v[pl.ds(PAD + base, L)]
        pprev = press_v[pl.ds(PAD + base - 1, L)]
        pon = pcur > pprev
        e_v[pl.ds(base, L)] = jnp.where(pon, end, 0)
        return jnp.maximum(carry, jnp.max(negpos))

    @plsc.parallel_loop(0, NCHUNK, unroll=8, carry=jnp.int32(0))
    def _fwd(i, carry):
        base = i * L
        e = e_v[pl.ds(base, L)]
        comb = jnp.maximum(plsc.cummax(e), carry)
        out_v[pl.ds(base, L)] = jnp.where(comb > base + lane, 1.0, 0.0)
        return jnp.maximum(carry, jnp.max(e))

    pltpu.sync_copy(out_v, out_hbm.at[w])


@jax.jit
def _run(x):
    mesh = plsc.VectorSubcoreMesh(core_axis_name="c", subcore_axis_name="s")
    f = pl.kernel(
        _sc_body,
        out_type=jax.ShapeDtypeStruct((32, T), jnp.float32),
        mesh=mesh,
        scratch_types=[
            pltpu.VMEM((BUF,), jnp.float32),
            pltpu.VMEM((BUF,), jnp.float32),
            pltpu.VMEM((T,), jnp.int32),
            pltpu.VMEM((T,), jnp.float32),
        ],
        compiler_params=pltpu.CompilerParams(
            needs_layout_passes=False, use_tc_tiling_on_sc=False
        ),
    )
    return f(x)


def kernel(gesture_labels):
    x = gesture_labels.reshape(64, T)
    return _run(x).reshape(16, 2, T)


# speedup vs baseline: 7.2065x; 1.1113x over previous
import functools

import jax
import jax.numpy as jnp
from jax import lax
from jax.experimental import pallas as pl
from jax.experimental.pallas import tpu as pltpu
from jax.experimental.pallas import tpu_sc as plsc

T = 4096
L = 16
NCHUNK = T // L
BIG = T + 10
PAD = 8
BUF = PAD + T + PAD


def _sc_body(in_hbm, out_hbm, press_v, rel_v, e_v, out_v, stage_v):
    b = lax.axis_index("s")
    f = lax.axis_index("c")

    pltpu.sync_copy(in_hbm.at[b, pl.ds(2 * f, 2), :], stage_v)

    lane = lax.iota(jnp.int32, L)

    press_v[pl.ds(0, L)] = jnp.zeros((L,), jnp.float32)
    rel_v[pl.ds(T, L)] = jnp.zeros((L,), jnp.float32)

    @plsc.parallel_loop(0, NCHUNK, unroll=8)
    def _untile(k):
        press_v[pl.ds(PAD + L * k, L)] = stage_v[0, pl.ds(L * k, L)]
        rel_v[pl.ds(PAD + L * k, L)] = stage_v[1, pl.ds(L * k, L)]

    @plsc.parallel_loop(0, NCHUNK, unroll=8, carry=jnp.int32(-BIG))
    def _bwd(i, carry):
        base = (NCHUNK - 1 - i) * L
        cur = rel_v[pl.ds(PAD + base, L)]
        nxt = rel_v[pl.ds(PAD + base + 1, L)]
        on = nxt > cur
        negpos = jnp.where(on, -(base + 1) - lane, -BIG)
        sm = jnp.flip(plsc.cummax(jnp.flip(negpos, 0)), 0)
        comb = jnp.maximum(sm, carry)
        end = jnp.minimum(8 - comb, T)
        pcur = press_v[pl.ds(PAD + base, L)]
        pprev = press_v[pl.ds(PAD + base - 1, L)]
        pon = pcur > pprev
        e_v[pl.ds(base, L)] = jnp.where(pon, end, 0)
        return jnp.maximum(carry, jnp.max(negpos))

    @plsc.parallel_loop(0, NCHUNK, unroll=8, carry=jnp.int32(0))
    def _fwd(i, carry):
        base = i * L
        e = e_v[pl.ds(base, L)]
        comb = jnp.maximum(plsc.cummax(e), carry)
        out_v[pl.ds(base, L)] = jnp.where(comb > base + lane, 1.0, 0.0)
        return jnp.maximum(carry, jnp.max(e))

    pltpu.sync_copy(out_v, out_hbm.at[b, f])


@jax.jit
def _run(x):
    mesh = plsc.VectorSubcoreMesh(core_axis_name="c", subcore_axis_name="s")
    f = pl.kernel(
        _sc_body,
        out_type=jax.ShapeDtypeStruct((16, 2, T), jnp.float32),
        mesh=mesh,
        scratch_types=[
            pltpu.VMEM((BUF,), jnp.float32),
            pltpu.VMEM((BUF,), jnp.float32),
            pltpu.VMEM((T,), jnp.int32),
            pltpu.VMEM((T,), jnp.float32),
            pltpu.VMEM((2, T), jnp.float32),
        ],
        compiler_params=pltpu.CompilerParams(
            needs_layout_passes=False, use_tc_tiling_on_sc=True
        ),
    )
    return f(x)


def kernel(gesture_labels):
    return _run(gesture_labels)
